# baseline (speedup 1.0000x reference)
import jax
import jax.numpy as jnp
from jax.experimental import pallas as pl
from jax.experimental.pallas import tpu as pltpu

_T = 2048
_HID = 2048
_QLR = 512
_NH = 16
_HD = 128
_RD = 64
_TOPK = 512
_FP8_MAX = 448.0

_RP = 256
_TB = 128
_SCHUNK = 512


def _pow2_ceil_scale(y):
    bits = jax.lax.bitcast_convert_type(y, jnp.int32)
    e = (bits >> 23) & 0xFF
    man = bits & 0x7FFFFF
    ceil_log2 = e - 127 + (man != 0).astype(jnp.int32)
    return jax.lax.bitcast_convert_type((ceil_log2 + 127) << 23, jnp.float32)


def _quant_fp8(x):
    amax = jnp.max(jnp.abs(x), axis=-1, keepdims=True)
    amax = jnp.maximum(amax, 1e-8)
    scale = _pow2_ceil_scale(amax / _FP8_MAX)
    xq = jnp.clip(x / scale, -_FP8_MAX, _FP8_MAX)
    xq8 = xq.astype(jnp.float8_e4m3fn).astype(jnp.bfloat16)
    return xq8, scale


def _prep_k_body(hs_ref, cos_ref, sin_ref, wk_ref, g_ref, b_ref,
                 wp_ref, bp_ref, k_out_ref, w_out_ref):
    hs = hs_ref[...]
    cos = cos_ref[...]
    sin = sin_ref[...]
    k = jax.lax.dot_general(hs, wk_ref[...], (((1,), (0,)), ((), ())),
                            preferred_element_type=jnp.float32)
    mu = jnp.mean(k, axis=-1, keepdims=True)
    var = jnp.mean((k - mu) ** 2, axis=-1, keepdims=True)
    k = (k - mu) / jnp.sqrt(var + 1e-6) * g_ref[...] + b_ref[...]
    k1 = k[:, : _RD // 2]
    k2 = k[:, _RD // 2 : _RD]
    k = jnp.concatenate(
        [k1 * cos - k2 * sin, k2 * cos + k1 * sin, k[:, _RD:]], axis=1)
    k8, k_scale = _quant_fp8(k)
    k_out_ref[...] = k8 * k_scale.astype(jnp.bfloat16)

    w = jax.lax.dot_general(hs, wp_ref[...], (((1,), (0,)), ((), ())),
                            preferred_element_type=jnp.float32)
    w_out_ref[...] = w + bp_ref[...]


def _prep_q_body(qr_ref, cos_ref, sin_ref, wqb_ref, wraw_ref,
                 q_out_ref, w_out_ref):
    cos = cos_ref[...]
    sin = sin_ref[...]
    q = jax.lax.dot_general(qr_ref[...], wqb_ref[...], (((1,), (0,)), ((), ())),
                            preferred_element_type=jnp.float32)
    q3 = q.reshape(_RP, _NH, _HD)
    c3 = cos[:, None, :]
    s3 = sin[:, None, :]
    q1 = q3[..., : _RD // 2]
    q2 = q3[..., _RD // 2 : _RD]
    q3 = jnp.concatenate(
        [q1 * c3 - q2 * s3, q2 * c3 + q1 * s3, q3[..., _RD:]], axis=-1)
    q8, q_scale = _quant_fp8(q3)
    q_out_ref[...] = q8.reshape(_RP * _NH, _HD)
    w = wraw_ref[...] * q_scale[..., 0]
    w = w * (_HD ** -0.5)
    w_out_ref[...] = w * (_NH ** -0.5)


def _cmpex(v, ix, pos, j, dirmask):
    n = v.shape[1]
    is_low = (pos & j) == 0
    pv = jnp.where(is_low, pltpu.roll(v, n - j, axis=1), pltpu.roll(v, j, axis=1))
    pi = jnp.where(is_low, pltpu.roll(ix, n - j, axis=1), pltpu.roll(ix, j, axis=1))
    av = jnp.where(is_low, v, pv)
    bv = jnp.where(is_low, pv, v)
    if dirmask is None:
        sw = av < bv
    else:
        sw = (dirmask & (av < bv)) | (~dirmask & (av > bv))
    return jnp.where(sw, pv, v), jnp.where(sw, pi, ix)


def _bitonic_cleanup(v, ix, pos, dirmask, first_j):
    nst = first_j.bit_length()

    def body(m, c):
        j = jax.lax.shift_right_logical(jnp.int32(first_j), m)
        return _cmpex(c[0], c[1], pos, j, dirmask)

    return jax.lax.fori_loop(0, nst, body, (v, ix))


def _topk512(v, ix):
    k = _TOPK
    pos = jax.lax.broadcasted_iota(jnp.int32, v.shape, 1)

    def outer(l, c):
        size = jax.lax.shift_left(jnp.int32(1), l)
        dirmask = (pos & size) == 0

        def inner(m, c2):
            j = jax.lax.shift_right_logical(size, m + 1)
            return _cmpex(c2[0], c2[1], pos, j, dirmask)

        return jax.lax.fori_loop(0, l, inner, c)

    v, ix = jax.lax.fori_loop(1, 10, outer, (v, ix))

    av = jnp.concatenate([v[:, 0:k], v[:, 2 * k : 3 * k]], axis=1)
    bv = jnp.concatenate([v[:, k : 2 * k], v[:, 3 * k :]], axis=1)
    ai = jnp.concatenate([ix[:, 0:k], ix[:, 2 * k : 3 * k]], axis=1)
    bi = jnp.concatenate([ix[:, k : 2 * k], ix[:, 3 * k :]], axis=1)
    ge = av >= bv
    mv = jnp.where(ge, av, bv)
    mi = jnp.where(ge, ai, bi)
    pos2 = jax.lax.broadcasted_iota(jnp.int32, mv.shape, 1)
    mv, mi = _bitonic_cleanup(mv, mi, pos2, (pos2 & k) == 0, k // 2)

    av, bv = mv[:, :k], mv[:, k:]
    ai, bi = mi[:, :k], mi[:, k:]
    ge = av >= bv
    fv = jnp.where(ge, av, bv)
    fi = jnp.where(ge, ai, bi)
    pos3 = jax.lax.broadcasted_iota(jnp.int32, fv.shape, 1)
    return _bitonic_cleanup(fv, fi, pos3, None, k // 2)


def _main_body(q_ref, k_ref, w_ref, idx_ref, val_ref):
    i = pl.program_id(0)
    qb = q_ref[...]
    kb = k_ref[...]
    w3 = w_ref[...][:, :, None].astype(jnp.bfloat16).astype(jnp.float32)
    chunks = []
    for c in range(_T // _SCHUNK):
        kc = kb[c * _SCHUNK : (c + 1) * _SCHUNK, :]
        logits = jax.lax.dot_general(
            qb, kc, (((1,), (1,)), ((), ())),
            preferred_element_type=jnp.float32)
        r = jnp.maximum(logits, 0.0).reshape(_TB, _NH, _SCHUNK)
        r = r.astype(jnp.bfloat16).astype(jnp.float32)
        p = r * w3
        t = [p[:, h] for h in range(_NH)]
        while len(t) > 1:
            t = [t[i] + t[i + 1] for i in range(0, len(t), 2)]
        chunks.append(t[0])
    scores = jnp.concatenate(chunks, axis=1)
    row = i * _TB + jax.lax.broadcasted_iota(jnp.int32, (_TB, _T), 0)
    col = jax.lax.broadcasted_iota(jnp.int32, (_TB, _T), 1)
    scores = jnp.where(col <= row, scores, -jnp.inf)
    fv, fi = _topk512(scores, col[:, : _T])
    val_ref[...] = fv
    idx_ref[...] = jnp.where(fv == -jnp.inf, -1, fi)


def kernel(hidden_states, qr, positions, Wq_b, Wk, k_norm_g, k_norm_b, Wp, bp):
    half = _RD // 2
    inv_freq = 1.0 / (10000.0 ** (jnp.arange(half, dtype=jnp.float32) / half))
    fr = positions.astype(jnp.float32)[:, None] * inv_freq[None, :]
    cos = jnp.cos(fr)
    sin = jnp.sin(fr)

    hs16 = hidden_states.astype(jnp.bfloat16)
    qr16 = qr.astype(jnp.bfloat16)
    wk16 = Wk.astype(jnp.bfloat16)
    wp16 = Wp.astype(jnp.bfloat16)
    wqb16 = Wq_b.astype(jnp.bfloat16)

    k_bf, w_raw = pl.pallas_call(
        _prep_k_body,
        grid=(1,),
        in_specs=[
            pl.BlockSpec((_T, _HID), lambda i: (0, 0)),
            pl.BlockSpec((_T, half), lambda i: (0, 0)),
            pl.BlockSpec((_T, half), lambda i: (0, 0)),
            pl.BlockSpec((_HID, _HD), lambda i: (0, 0)),
            pl.BlockSpec((1, _HD), lambda i: (0, 0)),
            pl.BlockSpec((1, _HD), lambda i: (0, 0)),
            pl.BlockSpec((_HID, _NH), lambda i: (0, 0)),
            pl.BlockSpec((1, _NH), lambda i: (0, 0)),
        ],
        out_specs=[
            pl.BlockSpec((_T, _HD), lambda i: (0, 0)),
            pl.BlockSpec((_T, _NH), lambda i: (0, 0)),
        ],
        out_shape=[
            jax.ShapeDtypeStruct((_T, _HD), jnp.bfloat16),
            jax.ShapeDtypeStruct((_T, _NH), jnp.float32),
        ],
    )(hs16, cos, sin, wk16,
      k_norm_g.reshape(1, _HD), k_norm_b.reshape(1, _HD),
      wp16, bp.reshape(1, _NH))

    q_bf, wts = pl.pallas_call(
        _prep_q_body,
        grid=(_T // _RP,),
        in_specs=[
            pl.BlockSpec((_RP, _QLR), lambda i: (i, 0)),
            pl.BlockSpec((_RP, half), lambda i: (i, 0)),
            pl.BlockSpec((_RP, half), lambda i: (i, 0)),
            pl.BlockSpec((_QLR, _NH * _HD), lambda i: (0, 0)),
            pl.BlockSpec((_RP, _NH), lambda i: (i, 0)),
        ],
        out_specs=[
            pl.BlockSpec((_RP * _NH, _HD), lambda i: (i, 0)),
            pl.BlockSpec((_RP, _NH), lambda i: (i, 0)),
        ],
        out_shape=[
            jax.ShapeDtypeStruct((_T * _NH, _HD), jnp.bfloat16),
            jax.ShapeDtypeStruct((_T, _NH), jnp.float32),
        ],
        compiler_params=pltpu.CompilerParams(
            dimension_semantics=("arbitrary",)),
    )(qr16, cos, sin, wqb16, w_raw)

    idx, vals = pl.pallas_call(
        _main_body,
        grid=(_T // _TB,),
        in_specs=[
            pl.BlockSpec((_TB * _NH, _HD), lambda i: (i, 0)),
            pl.BlockSpec((_T, _HD), lambda i: (0, 0)),
            pl.BlockSpec((_TB, _NH), lambda i: (i, 0)),
        ],
        out_specs=[
            pl.BlockSpec((_TB, _TOPK), lambda i: (i, 0)),
            pl.BlockSpec((_TB, _TOPK), lambda i: (i, 0)),
        ],
        out_shape=[
            jax.ShapeDtypeStruct((_T, _TOPK), jnp.int32),
            jax.ShapeDtypeStruct((_T, _TOPK), jnp.float32),
        ],
        compiler_params=pltpu.CompilerParams(
            dimension_semantics=("arbitrary",)),
    )(q_bf, k_bf, wts)
    return idx, vals
